# baseline (speedup 1.0000x reference)
import functools

import jax
import jax.numpy as jnp
from jax import lax
from jax.experimental import pallas as pl
from jax.experimental.pallas import tpu as pltpu
from jax.experimental.pallas import tpu_sc as plsc

F = 26
V = 100000
D = 32
B = 4096
NC = 2
NS = 16
NW = NC * NS
BPW = B // NW
LANES = 16


def _emb_body(cat_hbm, tab_hbm, out_hbm, idx_v, rows_v, sem):
    c = lax.axis_index("c")
    s = lax.axis_index("s")
    wid = s * NC + c
    b0 = wid * BPW

    pltpu.sync_copy(cat_hbm.at[:, pl.ds(b0, BPW)], idx_v)

    def off_body(i, carry):
        f = i >> 3
        k = i & 7
        sl = idx_v[f, pl.ds(k * LANES, LANES)]
        idx_v[f, pl.ds(k * LANES, LANES)] = sl + f * V
        return carry

    lax.fori_loop(0, F * (BPW // LANES), off_body, 0)

    def g_body(f, carry):
        pltpu.async_copy(tab_hbm.at[idx_v.at[f]], rows_v, sem).wait()
        pltpu.sync_copy(rows_v, out_hbm.at[pl.ds(b0, BPW), f])
        return carry

    lax.fori_loop(0, F, g_body, 0)


@jax.jit
def _emb_call(cat_features, tab_flat):
    mesh = plsc.VectorSubcoreMesh(core_axis_name="c", subcore_axis_name="s")
    run = pl.kernel(
        _emb_body,
        out_type=jax.ShapeDtypeStruct((B, F, D), jnp.float32),
        mesh=mesh,
        scratch_types=[
            pltpu.VMEM((F, BPW), jnp.int32),
            pltpu.VMEM((BPW, D), jnp.float32),
            pltpu.SemaphoreType.DMA,
        ],
        compiler_params=pltpu.CompilerParams(use_tc_tiling_on_sc=False),
    )
    return run(cat_features, tab_flat)


def kernel(cat_features, tables):
    cat = cat_features.astype(jnp.int32)
    tab_flat = tables.reshape(F * V, D)
    out = _emb_call(cat, tab_flat)
    return out.reshape(B, F * D)


# speedup vs baseline: 7.8609x; 7.8609x over previous
import jax
import jax.numpy as jnp
from jax import lax
from jax.experimental import pallas as pl
from jax.experimental.pallas import tpu as pltpu
from jax.experimental.pallas import tpu_sc as plsc

F = 26
V = 100000
D = 32
B = 4096
NC = 2
NS = 16
NW = NC * NS
LANES = 16
H0 = 50048
H1 = V - H0
NG = B // LANES


def _emb_body(cat_hbm, tab_hbm, out_hbm, idx_v, row_a, row_b, out_v,
              sem_i, sem_a, sem_b, sem_o0, sem_o1):
    c_ax = lax.axis_index("c")
    s_ax = lax.axis_index("s")
    w = s_ax * NC + c_ax

    def rot(j):
        return lax.rem(j + (w >> 3), F)

    f0 = rot(0)
    pltpu.async_copy(cat_hbm.at[f0], idx_v.at[0], sem_i)
    pltpu.async_copy(tab_hbm.at[f0, w, pl.ds(0, H0)], row_a, sem_a)

    def step(j, carry):
        jp = j & 1
        f = rot(j)
        ch = f * D + w

        pltpu.async_copy(tab_hbm.at[f, w, pl.ds(H0, H1)], row_b, sem_b)
        pltpu.make_async_copy(cat_hbm.at[f], idx_v.at[jp], sem_i).wait()
        pltpu.make_async_copy(tab_hbm.at[f, w, pl.ds(0, H0)], row_a, sem_a).wait()

        @pl.when(jnp.logical_and(j >= 2, jp == 0))
        def _():
            pltpu.make_async_copy(out_v.at[0], out_hbm.at[ch], sem_o0).wait()

        @pl.when(jnp.logical_and(j >= 2, jp == 1))
        def _():
            pltpu.make_async_copy(out_v.at[1], out_hbm.at[ch], sem_o1).wait()

        def g0(k, carry2):
            iv = idx_v[jp, pl.ds(k * LANES, LANES)]
            out_v[jp, pl.ds(k * LANES, LANES)] = plsc.load_gather(
                row_a, [jnp.minimum(iv, H0 - 1)])
            return carry2

        lax.fori_loop(0, NG, g0, 0)

        @pl.when(j < F - 1)
        def _():
            fn = rot(j + 1)
            pltpu.async_copy(cat_hbm.at[fn], idx_v.at[1 - jp], sem_i)
            pltpu.async_copy(tab_hbm.at[fn, w, pl.ds(0, H0)], row_a, sem_a)

        pltpu.make_async_copy(tab_hbm.at[f, w, pl.ds(H0, H1)], row_b, sem_b).wait()

        def g1(k, carry2):
            iv = idx_v[jp, pl.ds(k * LANES, LANES)]
            g = plsc.load_gather(row_b, [jnp.maximum(iv - H0, 0)])
            prev = out_v[jp, pl.ds(k * LANES, LANES)]
            out_v[jp, pl.ds(k * LANES, LANES)] = jnp.where(iv >= H0, g, prev)
            return carry2

        lax.fori_loop(0, NG, g1, 0)

        @pl.when(jp == 0)
        def _():
            pltpu.async_copy(out_v.at[0], out_hbm.at[ch], sem_o0)

        @pl.when(jp == 1)
        def _():
            pltpu.async_copy(out_v.at[1], out_hbm.at[ch], sem_o1)

        return carry

    lax.fori_loop(0, F, step, 0)

    ch24 = rot(F - 2) * D + w
    ch25 = rot(F - 1) * D + w
    pltpu.make_async_copy(out_v.at[0], out_hbm.at[ch24], sem_o0).wait()
    pltpu.make_async_copy(out_v.at[1], out_hbm.at[ch25], sem_o1).wait()


@jax.jit
def _emb_call(cat_features, tab_t):
    mesh = plsc.VectorSubcoreMesh(core_axis_name="c", subcore_axis_name="s")
    run = pl.kernel(
        _emb_body,
        out_type=jax.ShapeDtypeStruct((F * D, B), jnp.float32),
        mesh=mesh,
        scratch_types=[
            pltpu.VMEM((2, B), jnp.int32),
            pltpu.VMEM((H0,), jnp.float32),
            pltpu.VMEM((H1,), jnp.float32),
            pltpu.VMEM((2, B), jnp.float32),
            pltpu.SemaphoreType.DMA,
            pltpu.SemaphoreType.DMA,
            pltpu.SemaphoreType.DMA,
            pltpu.SemaphoreType.DMA,
            pltpu.SemaphoreType.DMA,
        ],
        compiler_params=pltpu.CompilerParams(
            use_tc_tiling_on_sc=True, needs_layout_passes=False
        ),
    )
    return run(cat_features, tab_t)


def kernel(cat_features, tables):
    cat = cat_features.astype(jnp.int32)
    tab_t = jnp.transpose(tables, (0, 2, 1))
    out = _emb_call(cat, tab_t)
    return jnp.transpose(out, (1, 0)).reshape(B, F * D)
